# baseline (speedup 1.0000x reference)
import dataclasses
import functools

import jax
import jax.numpy as jnp
from jax import lax
from jax.experimental import pallas as pl
from jax.experimental.pallas import tpu as pltpu
from jax.experimental.pallas import tpu_sc as plsc

N = 10000
D = 128
E = 320000

NC = 2
NS = 16
NW = NC * NS
RW = D // NW
CE = 4096
CH = (E + NW * CE - 1) // (NW * CE) * NW
EP = CH * CE
EPW = EP // NW
NP = 10240

_f32 = jnp.float32


def _sc_params():
    cp = pltpu.CompilerParams()
    if "needs_layout_passes" in pltpu.CompilerParams.__dataclass_fields__:
        cp = dataclasses.replace(cp, needs_layout_passes=False)
    return cp


def _vector_mesh():
    return plsc.VectorSubcoreMesh(core_axis_name="c", subcore_axis_name="s")


def _degree_kernel(src_p, dst_p):
    @functools.partial(
        pl.kernel,
        out_type=[
            jax.ShapeDtypeStruct((NW, NP), _f32),
            jax.ShapeDtypeStruct((NW, NP), _f32),
        ],
        mesh=_vector_mesh(),
        scratch_types=[
            pltpu.VMEM((NP,), _f32),
            pltpu.VMEM((NP,), _f32),
            pltpu.VMEM((EPW,), jnp.int32),
            pltpu.VMEM((EPW,), jnp.int32),
            pltpu.SemaphoreType.DMA,
        ],
        compiler_params=_sc_params(),
    )
    def k(src_hbm, dst_hbm, dgo_hbm, dgi_hbm, dgo_v, dgi_v, sidx_v, didx_v, sem):
        cid = lax.axis_index("c")
        sid = lax.axis_index("s")
        w = cid * NS + sid

        pltpu.async_copy(src_hbm.at[pl.ds(w * EPW, EPW)], sidx_v, sem).wait()
        pltpu.async_copy(dst_hbm.at[pl.ds(w * EPW, EPW)], didx_v, sem).wait()

        @pl.loop(0, NP // 16)
        def _(i):
            dgo_v[pl.ds(i * 16, 16)] = jnp.zeros((16,), _f32)
            dgi_v[pl.ds(i * 16, 16)] = jnp.zeros((16,), _f32)

        ones16 = jnp.ones((16,), _f32)

        @pl.loop(0, EPW // 16)
        def _(g):
            s16 = sidx_v[pl.ds(g * 16, 16)]
            d16 = didx_v[pl.ds(g * 16, 16)]
            plsc.addupdate_scatter(dgo_v, [s16], ones16)
            plsc.addupdate_scatter(dgi_v, [d16], ones16)

        pltpu.sync_copy(dgo_v, dgo_hbm.at[w])
        pltpu.sync_copy(dgi_v, dgi_hbm.at[w])

    return k(src_p, dst_p)


def _agg_kernel(xT, src_p, dst_p):
    @functools.partial(
        pl.kernel,
        out_type=jax.ShapeDtypeStruct((D, NP), _f32),
        mesh=_vector_mesh(),
        scratch_types=[
            pltpu.VMEM((RW, N), _f32),
            pltpu.VMEM((RW, NP), _f32),
            pltpu.VMEM((2, CE), jnp.int32),
            pltpu.VMEM((2, CE), jnp.int32),
            pltpu.SemaphoreType.DMA,
            pltpu.SemaphoreType.DMA,
            pltpu.SemaphoreType.DMA,
        ],
        compiler_params=_sc_params(),
    )
    def k(xT_hbm, src_hbm, dst_hbm, out_hbm, tab_v, acc_v, sidx_v, didx_v,
          sem_t, sem_s, sem_d):
        cid = lax.axis_index("c")
        sid = lax.axis_index("s")
        w = cid * NS + sid

        tab_cp = pltpu.async_copy(xT_hbm.at[pl.ds(w * RW, RW)], tab_v, sem_t)

        pltpu.async_copy(src_hbm.at[pl.ds(0, CE)], sidx_v.at[0], sem_s).wait()
        pltpu.async_copy(dst_hbm.at[pl.ds(0, CE)], didx_v.at[0], sem_d).wait()

        @pl.loop(0, RW)
        def _(r):
            @pl.loop(0, NP // 16)
            def _(i):
                acc_v[r, pl.ds(i * 16, 16)] = jnp.zeros((16,), _f32)

        tab_cp.wait()

        UG = 16

        def do_chunk(buf):
            @plsc.parallel_loop(0, CE // (16 * UG), unroll=1)
            def _(g):
                svecs = []
                dvecs = []
                for u in range(UG):
                    off = pl.ds((g * UG + u) * 16, 16)
                    svecs.append(sidx_v[buf, off])
                    dvecs.append(didx_v[buf, off])
                vals = []
                for u in range(UG):
                    for r in range(RW):
                        r16 = jnp.full((16,), r, jnp.int32)
                        vals.append(plsc.load_gather(tab_v, [r16, svecs[u]]))
                for u in range(UG):
                    for r in range(RW):
                        r16 = jnp.full((16,), r, jnp.int32)
                        plsc.addupdate_scatter(acc_v, [r16, dvecs[u]],
                                               vals[u * RW + r])

        @pl.loop(0, CH // 2)
        def _(jj):
            j = jj * 2
            nxt = (j + 1) * CE
            cp_s = pltpu.async_copy(src_hbm.at[pl.ds(nxt, CE)], sidx_v.at[1], sem_s)
            cp_d = pltpu.async_copy(dst_hbm.at[pl.ds(nxt, CE)], didx_v.at[1], sem_d)
            do_chunk(0)
            cp_s.wait()
            cp_d.wait()

            @pl.when(j + 2 < CH)
            def _():
                nxt2 = (j + 2) * CE
                cp_s2 = pltpu.async_copy(src_hbm.at[pl.ds(nxt2, CE)],
                                         sidx_v.at[0], sem_s)
                cp_d2 = pltpu.async_copy(dst_hbm.at[pl.ds(nxt2, CE)],
                                         didx_v.at[0], sem_d)
                do_chunk(1)
                cp_s2.wait()
                cp_d2.wait()

            @pl.when(j + 2 >= CH)
            def _():
                do_chunk(1)

        if CH % 2 == 1:
            pltpu.async_copy(src_hbm.at[pl.ds((CH - 1) * CE, CE)],
                             sidx_v.at[0], sem_s).wait()
            pltpu.async_copy(dst_hbm.at[pl.ds((CH - 1) * CE, CE)],
                             didx_v.at[0], sem_d).wait()
            do_chunk(0)

        pltpu.sync_copy(acc_v, out_hbm.at[pl.ds(w * RW, RW)])

    return k(xT, src_p, dst_p)


def _norms_kernel(dgo_p, dgi_p, features):
    def body(dgo_ref, dgi_ref, x_ref, xsT_ref, ndr_ref, nsr_ref):
        do = jnp.sum(dgo_ref[...], axis=0, keepdims=True)[:, :N]
        di = jnp.sum(dgi_ref[...], axis=0, keepdims=True)[:, :N]
        ns = jnp.where(do > 0, lax.rsqrt(jnp.maximum(do, 1e-12)), 0.0)
        nd = jnp.where(di > 0, lax.rsqrt(jnp.maximum(di, 1e-12)), 0.0)
        nsr_ref[...] = ns
        ndr_ref[...] = nd
        xsT_ref[...] = jnp.transpose(x_ref[...]) * ns

    return pl.pallas_call(
        body,
        out_shape=[
            jax.ShapeDtypeStruct((D, N), _f32),
            jax.ShapeDtypeStruct((1, N), _f32),
            jax.ShapeDtypeStruct((1, N), _f32),
        ],
    )(dgo_p, dgi_p, features)


def _layer_kernel(aggT, ndr, nsr, Wt, b, last):
    def body(p_ref, nd_ref, ns_ref, w_ref, b_ref, o_ref):
        t = p_ref[:, :N] * nd_ref[...]
        yT = jnp.dot(w_ref[...], t, preferred_element_type=_f32,
                     precision=lax.Precision.HIGHEST) + b_ref[...]
        if last:
            o_ref[...] = jnp.transpose(yT)
        else:
            o_ref[...] = jnp.maximum(yT, 0.0) * ns_ref[...]

    out_shape = (jax.ShapeDtypeStruct((N, D), _f32) if last
                 else jax.ShapeDtypeStruct((D, N), _f32))
    return pl.pallas_call(body, out_shape=out_shape)(aggT, ndr, nsr, Wt, b)


def kernel(features, edge_index, W0, b0, W1, b1, W2, b2):
    src = edge_index[0]
    dst = edge_index[1]
    pad = EP - E
    pad_dst = (N + (jnp.arange(pad, dtype=jnp.int32) % (NP - N))).astype(jnp.int32)
    src_p = jnp.concatenate([src, jnp.zeros((pad,), jnp.int32)])
    dst_p = jnp.concatenate([dst, pad_dst])

    dgo_p, dgi_p = _degree_kernel(src_p, dst_p)
    hT, ndr, nsr = _norms_kernel(dgo_p, dgi_p, features)

    for W, b, last in ((W0, b0, False), (W1, b1, False), (W2, b2, True)):
        aggT = _agg_kernel(hT, src_p, dst_p)
        hT = _layer_kernel(aggT, ndr, nsr, W.T, b.reshape(D, 1), last)
    return hT


# speedup vs baseline: 1.3330x; 1.3330x over previous
import dataclasses
import functools

import jax
import jax.numpy as jnp
from jax import lax
from jax.experimental import pallas as pl
from jax.experimental.pallas import tpu as pltpu
from jax.experimental.pallas import tpu_sc as plsc

N = 10000
D = 128
E = 320000

NC = 2
NS = 16
NW = NC * NS
RW = D // NW
CE = 4096
CH = (E + NW * CE - 1) // (NW * CE) * NW
EP = CH * CE
EPW = EP // NW
NP = 10240

_f32 = jnp.float32


def _sc_params():
    cp = pltpu.CompilerParams()
    if "needs_layout_passes" in pltpu.CompilerParams.__dataclass_fields__:
        cp = dataclasses.replace(cp, needs_layout_passes=False)
    return cp


def _vector_mesh():
    return plsc.VectorSubcoreMesh(core_axis_name="c", subcore_axis_name="s")


def _degree_kernel(src_p, dst_p):
    @functools.partial(
        pl.kernel,
        out_type=[
            jax.ShapeDtypeStruct((NW, NP), _f32),
            jax.ShapeDtypeStruct((NW, NP), _f32),
        ],
        mesh=_vector_mesh(),
        scratch_types=[
            pltpu.VMEM((NP,), _f32),
            pltpu.VMEM((NP,), _f32),
            pltpu.VMEM((EPW,), jnp.int32),
            pltpu.VMEM((EPW,), jnp.int32),
            pltpu.SemaphoreType.DMA,
        ],
        compiler_params=_sc_params(),
    )
    def k(src_hbm, dst_hbm, dgo_hbm, dgi_hbm, dgo_v, dgi_v, sidx_v, didx_v, sem):
        cid = lax.axis_index("c")
        sid = lax.axis_index("s")
        w = cid * NS + sid

        pltpu.async_copy(src_hbm.at[pl.ds(w * EPW, EPW)], sidx_v, sem).wait()
        pltpu.async_copy(dst_hbm.at[pl.ds(w * EPW, EPW)], didx_v, sem).wait()

        @pl.loop(0, NP // 16)
        def _(i):
            dgo_v[pl.ds(i * 16, 16)] = jnp.zeros((16,), _f32)
            dgi_v[pl.ds(i * 16, 16)] = jnp.zeros((16,), _f32)

        ones16 = jnp.ones((16,), _f32)

        @pl.loop(0, EPW // 16)
        def _(g):
            s16 = sidx_v[pl.ds(g * 16, 16)]
            d16 = didx_v[pl.ds(g * 16, 16)]
            plsc.addupdate_scatter(dgo_v, [s16], ones16)
            plsc.addupdate_scatter(dgi_v, [d16], ones16)

        pltpu.sync_copy(dgo_v, dgo_hbm.at[w])
        pltpu.sync_copy(dgi_v, dgi_hbm.at[w])

    return k(src_p, dst_p)


def _agg_kernel(xT, src_p, dst_p):
    @functools.partial(
        pl.kernel,
        out_type=jax.ShapeDtypeStruct((D, NP), _f32),
        mesh=_vector_mesh(),
        scratch_types=[
            pltpu.VMEM((RW, N), _f32),
            pltpu.VMEM((RW, NP), _f32),
            pltpu.VMEM((2, CE), jnp.int32),
            pltpu.VMEM((2, CE), jnp.int32),
            pltpu.SemaphoreType.DMA,
            pltpu.SemaphoreType.DMA,
            pltpu.SemaphoreType.DMA,
        ],
        compiler_params=_sc_params(),
    )
    def k(xT_hbm, src_hbm, dst_hbm, out_hbm, tab_v, acc_v, sidx_v, didx_v,
          sem_t, sem_s, sem_d):
        cid = lax.axis_index("c")
        sid = lax.axis_index("s")
        w = cid * NS + sid

        tab_cp = pltpu.async_copy(xT_hbm.at[pl.ds(w * RW, RW)], tab_v, sem_t)

        pltpu.async_copy(src_hbm.at[pl.ds(0, CE)], sidx_v.at[0], sem_s).wait()
        pltpu.async_copy(dst_hbm.at[pl.ds(0, CE)], didx_v.at[0], sem_d).wait()

        @pl.loop(0, RW)
        def _(r):
            @pl.loop(0, NP // 16)
            def _(i):
                acc_v[r, pl.ds(i * 16, 16)] = jnp.zeros((16,), _f32)

        tab_cp.wait()

        UG = 4

        def do_chunk(buf):
            @plsc.parallel_loop(0, CE // (16 * UG), unroll=4)
            def _(g):
                svecs = []
                dvecs = []
                for u in range(UG):
                    off = pl.ds((g * UG + u) * 16, 16)
                    svecs.append(sidx_v[buf, off])
                    dvecs.append(didx_v[buf, off])
                vals = []
                for u in range(UG):
                    for r in range(RW):
                        r16 = jnp.full((16,), r, jnp.int32)
                        vals.append(plsc.load_gather(tab_v, [r16, svecs[u]]))
                for u in range(UG):
                    for r in range(RW):
                        r16 = jnp.full((16,), r, jnp.int32)
                        plsc.addupdate_scatter(acc_v, [r16, dvecs[u]],
                                               vals[u * RW + r])

        @pl.loop(0, CH // 2)
        def _(jj):
            j = jj * 2
            nxt = (j + 1) * CE
            cp_s = pltpu.async_copy(src_hbm.at[pl.ds(nxt, CE)], sidx_v.at[1], sem_s)
            cp_d = pltpu.async_copy(dst_hbm.at[pl.ds(nxt, CE)], didx_v.at[1], sem_d)
            do_chunk(0)
            cp_s.wait()
            cp_d.wait()

            @pl.when(j + 2 < CH)
            def _():
                nxt2 = (j + 2) * CE
                cp_s2 = pltpu.async_copy(src_hbm.at[pl.ds(nxt2, CE)],
                                         sidx_v.at[0], sem_s)
                cp_d2 = pltpu.async_copy(dst_hbm.at[pl.ds(nxt2, CE)],
                                         didx_v.at[0], sem_d)
                do_chunk(1)
                cp_s2.wait()
                cp_d2.wait()

            @pl.when(j + 2 >= CH)
            def _():
                do_chunk(1)

        if CH % 2 == 1:
            pltpu.async_copy(src_hbm.at[pl.ds((CH - 1) * CE, CE)],
                             sidx_v.at[0], sem_s).wait()
            pltpu.async_copy(dst_hbm.at[pl.ds((CH - 1) * CE, CE)],
                             didx_v.at[0], sem_d).wait()
            do_chunk(0)

        pltpu.sync_copy(acc_v, out_hbm.at[pl.ds(w * RW, RW)])

    return k(xT, src_p, dst_p)


def _norms_kernel(dgo_p, dgi_p, features):
    def body(dgo_ref, dgi_ref, x_ref, xsT_ref, ndr_ref, nsr_ref):
        do = jnp.sum(dgo_ref[...], axis=0, keepdims=True)[:, :N]
        di = jnp.sum(dgi_ref[...], axis=0, keepdims=True)[:, :N]
        ns = jnp.where(do > 0, lax.rsqrt(jnp.maximum(do, 1e-12)), 0.0)
        nd = jnp.where(di > 0, lax.rsqrt(jnp.maximum(di, 1e-12)), 0.0)
        nsr_ref[...] = ns
        ndr_ref[...] = nd
        xsT_ref[...] = jnp.transpose(x_ref[...]) * ns

    return pl.pallas_call(
        body,
        out_shape=[
            jax.ShapeDtypeStruct((D, N), _f32),
            jax.ShapeDtypeStruct((1, N), _f32),
            jax.ShapeDtypeStruct((1, N), _f32),
        ],
    )(dgo_p, dgi_p, features)


def _layer_kernel(aggT, ndr, nsr, Wt, b, last):
    def body(p_ref, nd_ref, ns_ref, w_ref, b_ref, o_ref):
        t = p_ref[:, :N] * nd_ref[...]
        yT = jnp.dot(w_ref[...], t, preferred_element_type=_f32,
                     precision=lax.Precision.HIGHEST) + b_ref[...]
        if last:
            o_ref[...] = jnp.transpose(yT)
        else:
            o_ref[...] = jnp.maximum(yT, 0.0) * ns_ref[...]

    out_shape = (jax.ShapeDtypeStruct((N, D), _f32) if last
                 else jax.ShapeDtypeStruct((D, N), _f32))
    return pl.pallas_call(body, out_shape=out_shape)(aggT, ndr, nsr, Wt, b)


def kernel(features, edge_index, W0, b0, W1, b1, W2, b2):
    src = edge_index[0]
    dst = edge_index[1]
    pad = EP - E
    pad_dst = (N + (jnp.arange(pad, dtype=jnp.int32) % (NP - N))).astype(jnp.int32)
    src_p = jnp.concatenate([src, jnp.zeros((pad,), jnp.int32)])
    dst_p = jnp.concatenate([dst, pad_dst])

    dgo_p, dgi_p = _degree_kernel(src_p, dst_p)
    hT, ndr, nsr = _norms_kernel(dgo_p, dgi_p, features)

    for W, b, last in ((W0, b0, False), (W1, b1, False), (W2, b2, True)):
        aggT = _agg_kernel(hT, src_p, dst_p)
        hT = _layer_kernel(aggT, ndr, nsr, W.T, b.reshape(D, 1), last)
    return hT
